# baseline (speedup 1.0000x reference)
import functools

import jax
import jax.numpy as jnp
from jax import lax
from jax.experimental import pallas as pl
from jax.experimental.pallas import tpu as pltpu
from jax.experimental.pallas import tpu_sc as plsc

DIM2 = 512
N_CODES = 8192
N_TOK = 8192
BN = 256
RB = 1024
N_TILES = N_CODES // BN
D = 256
_HALF_TILES = N_TILES // 2


def _argmin_body(fl_ref, fh_ref, fsq_ref, e_ref, esq_ref, out_ref,
                 fbf_ref, rv_ref, ri_ref, c0v_ref, c0i_ref):
    n = pl.program_id(0)
    lane = lax.broadcasted_iota(jnp.int32, (1, BN), 1)

    e = e_ref[...].astype(jnp.bfloat16)
    esq = esq_ref[...]
    for mi in range(N_TOK // RB):
        rows = pl.ds(mi * RB, RB)

        @pl.when(n == 0)
        def _cast():
            fbf_ref[rows, :D] = (2.0 * fl_ref[rows, :]).astype(jnp.bfloat16)
            fbf_ref[rows, D:] = (2.0 * fh_ref[rows, :]).astype(jnp.bfloat16)

        mm = jnp.dot(fbf_ref[rows, :], e, preferred_element_type=jnp.float32)
        d = (fsq_ref[rows, :] - mm) + esq
        fresh = (n == 0) | (n == _HALF_TILES)
        cur_v = jnp.where(fresh, jnp.inf, rv_ref[rows, :])
        cur_i = jnp.where(fresh, 0, ri_ref[rows, :])
        take = d < cur_v
        rv_ref[rows, :] = jnp.where(take, d, cur_v)
        ri_ref[rows, :] = jnp.where(take, n * BN + lane, cur_i)

    @pl.when((n == _HALF_TILES - 1) | (n == N_TILES - 1))
    def _reduce_chunk():
        for mi in range(N_TOK // RB):
            rows = pl.ds(mi * RB, RB)
            rv = rv_ref[rows, :]
            ri = ri_ref[rows, :]
            m = jnp.min(rv, axis=1, keepdims=True)
            ie = jnp.min(jnp.where(rv == m, ri, N_CODES), axis=1,
                         keepdims=True)
            @pl.when(n == _HALF_TILES - 1)
            def _save0():
                c0v_ref[rows, :] = m
                c0i_ref[rows, :] = ie

            @pl.when(n == N_TILES - 1)
            def _emit():
                m0b = c0v_ref[rows, :].astype(jnp.bfloat16).astype(jnp.float32)
                out_ref[rows, :] = jnp.where(m < m0b, ie, c0i_ref[rows, :])


def _fused_argmin(flatten_lr, flatten_hr, fsq, embed, esq):
    return pl.pallas_call(
        _argmin_body,
        grid=(N_TILES,),
        in_specs=[
            pl.BlockSpec((N_TOK, D), lambda n: (0, 0)),
            pl.BlockSpec((N_TOK, D), lambda n: (0, 0)),
            pl.BlockSpec((N_TOK, 1), lambda n: (0, 0)),
            pl.BlockSpec((DIM2, BN), lambda n: (0, n)),
            pl.BlockSpec((1, BN), lambda n: (0, n)),
        ],
        out_specs=pl.BlockSpec((N_TOK, 1), lambda n: (0, 0)),
        out_shape=jax.ShapeDtypeStruct((N_TOK, 1), jnp.int32),
        scratch_shapes=[
            pltpu.VMEM((N_TOK, DIM2), jnp.bfloat16),
            pltpu.VMEM((N_TOK, BN), jnp.float32),
            pltpu.VMEM((N_TOK, BN), jnp.int32),
            pltpu.VMEM((N_TOK, 1), jnp.float32),
            pltpu.VMEM((N_TOK, 1), jnp.int32),
        ],
    )(flatten_lr, flatten_hr, fsq, embed, esq)


_CHUNK = 64
_NCHUNK = 4


@functools.lru_cache(maxsize=1)
def _build_sc_gather():
    mesh = plsc.VectorSubcoreMesh(core_axis_name="c", subcore_axis_name="s")

    @functools.partial(
        pl.kernel,
        out_type=jax.ShapeDtypeStruct((N_TOK, DIM2), jnp.float32),
        mesh=mesh,
        scratch_types=[
            pltpu.VMEM((_NCHUNK, _CHUNK), jnp.int32),
            pltpu.VMEM((_CHUNK, DIM2), jnp.float32),
            pltpu.VMEM((_CHUNK, DIM2), jnp.float32),
            pltpu.SemaphoreType.DMA,
            pltpu.SemaphoreType.DMA,
        ],
    )
    def _sc_gather(idx_hbm, tab_hbm, out_hbm,
                   idx_v, rows_a, rows_b, sem_a, sem_b):
        wid = lax.axis_index("s") * 2 + lax.axis_index("c")
        base = wid * (_NCHUNK * _CHUNK)
        pltpu.sync_copy(idx_hbm.at[pl.ds(wid * _NCHUNK, _NCHUNK)], idx_v)
        bufs = (rows_a, rows_b)
        sems = (sem_a, sem_b)
        cps = [None, None]
        for k in range(_NCHUNK):
            b = k & 1
            cps[b] = pltpu.async_copy(tab_hbm.at[idx_v.at[k]], bufs[b], sems[b])
            if k >= 1:
                pb = (k - 1) & 1
                cps[pb].wait()
                pltpu.sync_copy(
                    bufs[pb],
                    out_hbm.at[pl.ds(base + (k - 1) * _CHUNK, _CHUNK)])
        lb = (_NCHUNK - 1) & 1
        cps[lb].wait()
        pltpu.sync_copy(
            bufs[lb],
            out_hbm.at[pl.ds(base + (_NCHUNK - 1) * _CHUNK, _CHUNK)])

    return _sc_gather


_K3_RB = 1024
_K3_STEPS = N_TOK // _K3_RB
_INV_N = 1.0 / float(N_TOK * D)


def _st_body(q_ref, xl_ref, xh_ref, olr_ref, ohr_ref, slr_ref, shr_ref):
    i = pl.program_id(0)
    xl = xl_ref[...]
    xh = xh_ref[...]
    dl = q_ref[:, :D] - xl
    dh = q_ref[:, D:] - xh
    olr_ref[...] = xl + dl
    ohr_ref[...] = xh + dh
    sl = jnp.sum(dl * dl).reshape(1, 1)
    sh = jnp.sum(dh * dh).reshape(1, 1)
    acc_l = jnp.where(i == 0, sl, slr_ref[...] + sl)
    acc_h = jnp.where(i == 0, sh, shr_ref[...] + sh)
    last = i == _K3_STEPS - 1
    slr_ref[...] = jnp.where(last, acc_l * _INV_N, acc_l)
    shr_ref[...] = jnp.where(last, acc_h * _INV_N, acc_h)


def _straight_through(q, xl, xh):
    row_spec = pl.BlockSpec((_K3_RB, DIM2), lambda i: (i, 0))
    half_spec = pl.BlockSpec((_K3_RB, D), lambda i: (i, 0))
    one_spec = pl.BlockSpec((1, 1), lambda i: (0, 0))
    return pl.pallas_call(
        _st_body,
        grid=(_K3_STEPS,),
        in_specs=[row_spec, half_spec, half_spec],
        out_specs=[half_spec, half_spec, one_spec, one_spec],
        out_shape=[
            jax.ShapeDtypeStruct((N_TOK, D), jnp.float32),
            jax.ShapeDtypeStruct((N_TOK, D), jnp.float32),
            jax.ShapeDtypeStruct((1, 1), jnp.float32),
            jax.ShapeDtypeStruct((1, 1), jnp.float32),
        ],
    )(q, xl, xh)


def kernel(input_hr, input_lr, embed_lr, embed_hr):
    dim = input_hr.shape[-1]
    flatten_hr = input_hr.reshape(-1, dim)
    flatten_lr = input_lr.reshape(-1, dim)
    flatten = jnp.concatenate([flatten_lr, flatten_hr], axis=1)
    embed = jnp.concatenate([embed_lr, embed_hr], axis=0)
    fsq = (flatten ** 2).sum(1, keepdims=True)
    esq = (embed ** 2).sum(0, keepdims=True)

    ind2d = _fused_argmin(flatten_lr, flatten_hr, fsq, embed, esq)

    idx_hbm = ind2d.reshape(N_TOK // _CHUNK, _CHUNK)
    q = _build_sc_gather()(idx_hbm, embed.T)

    o_lr, o_hr, s_lr, s_hr = _straight_through(q, flatten_lr, flatten_hr)

    embed_ind = ind2d.reshape(input_hr.shape[:-1])
    quantize_hr = o_hr.reshape(input_hr.shape)
    quantize_lr = o_lr.reshape(input_lr.shape)
    diff_hr = s_hr.reshape(())
    diff_lr = s_lr.reshape(())
    return (quantize_hr, quantize_lr, diff_hr, diff_lr, embed_ind, embed_ind)


# speedup vs baseline: 1.2617x; 1.2617x over previous
import functools

import jax
import jax.numpy as jnp
from jax import lax
from jax.experimental import pallas as pl
from jax.experimental.pallas import tpu as pltpu
from jax.experimental.pallas import tpu_sc as plsc

DIM2 = 512
N_CODES = 8192
N_TOK = 8192
BN = 256
RB = 1024
N_TILES = N_CODES // BN
D = 256
_HALF_TILES = N_TILES // 2


def _argmin_body(f_ref, fsq_ref, e_ref, esq_ref, out_ref,
                 rv_ref, ri_ref, c0v_ref, c0i_ref):
    n = pl.program_id(0)
    lane = lax.broadcasted_iota(jnp.int32, (1, BN), 1)

    e = e_ref[...].astype(jnp.bfloat16)
    esq = esq_ref[...]
    for mi in range(N_TOK // RB):
        rows = pl.ds(mi * RB, RB)
        f = (2.0 * f_ref[rows, :]).astype(jnp.bfloat16)
        mm = jnp.dot(f, e, preferred_element_type=jnp.float32)
        d = (fsq_ref[rows, :] - mm) + esq
        fresh = (n == 0) | (n == _HALF_TILES)
        cur_v = jnp.where(fresh, jnp.inf, rv_ref[rows, :])
        cur_i = jnp.where(fresh, 0, ri_ref[rows, :])
        take = d < cur_v
        rv_ref[rows, :] = jnp.where(take, d, cur_v)
        ri_ref[rows, :] = jnp.where(take, n * BN + lane, cur_i)

    @pl.when((n == _HALF_TILES - 1) | (n == N_TILES - 1))
    def _reduce_chunk():
        for mi in range(N_TOK // RB):
            rows = pl.ds(mi * RB, RB)
            rv = rv_ref[rows, :]
            ri = ri_ref[rows, :]
            m = jnp.min(rv, axis=1, keepdims=True)
            ie = jnp.min(jnp.where(rv == m, ri, N_CODES), axis=1,
                         keepdims=True)
            @pl.when(n == _HALF_TILES - 1)
            def _save0():
                c0v_ref[rows, :] = m
                c0i_ref[rows, :] = ie

            @pl.when(n == N_TILES - 1)
            def _emit():
                m0b = c0v_ref[rows, :].astype(jnp.bfloat16).astype(jnp.float32)
                out_ref[rows, :] = jnp.where(m < m0b, ie, c0i_ref[rows, :])


def _fused_argmin(flatten, fsq, embed, esq):
    return pl.pallas_call(
        _argmin_body,
        grid=(N_TILES,),
        in_specs=[
            pl.BlockSpec((N_TOK, DIM2), lambda n: (0, 0)),
            pl.BlockSpec((N_TOK, 1), lambda n: (0, 0)),
            pl.BlockSpec((DIM2, BN), lambda n: (0, n)),
            pl.BlockSpec((1, BN), lambda n: (0, n)),
        ],
        out_specs=pl.BlockSpec((N_TOK, 1), lambda n: (0, 0)),
        out_shape=jax.ShapeDtypeStruct((N_TOK, 1), jnp.int32),
        scratch_shapes=[
            pltpu.VMEM((N_TOK, BN), jnp.float32),
            pltpu.VMEM((N_TOK, BN), jnp.int32),
            pltpu.VMEM((N_TOK, 1), jnp.float32),
            pltpu.VMEM((N_TOK, 1), jnp.int32),
        ],
    )(flatten, fsq, embed, esq)


_CHUNK = 64
_NCHUNK = 4


@functools.lru_cache(maxsize=1)
def _build_sc_gather():
    mesh = plsc.VectorSubcoreMesh(core_axis_name="c", subcore_axis_name="s")

    @functools.partial(
        pl.kernel,
        out_type=jax.ShapeDtypeStruct((N_TOK, DIM2), jnp.float32),
        mesh=mesh,
        scratch_types=[
            pltpu.VMEM((_NCHUNK, _CHUNK), jnp.int32),
            pltpu.VMEM((_CHUNK, DIM2), jnp.float32),
            pltpu.VMEM((_CHUNK, DIM2), jnp.float32),
            pltpu.SemaphoreType.DMA,
            pltpu.SemaphoreType.DMA,
        ],
    )
    def _sc_gather(idx_hbm, tab_hbm, out_hbm,
                   idx_v, rows_a, rows_b, sem_a, sem_b):
        wid = lax.axis_index("s") * 2 + lax.axis_index("c")
        base = wid * (_NCHUNK * _CHUNK)
        pltpu.sync_copy(idx_hbm.at[pl.ds(wid * _NCHUNK, _NCHUNK)], idx_v)
        bufs = (rows_a, rows_b)
        sems = (sem_a, sem_b)
        cps = [None, None]
        for k in range(_NCHUNK):
            b = k & 1
            cps[b] = pltpu.async_copy(tab_hbm.at[idx_v.at[k]], bufs[b], sems[b])
            if k >= 1:
                pb = (k - 1) & 1
                cps[pb].wait()
                pltpu.sync_copy(
                    bufs[pb],
                    out_hbm.at[pl.ds(base + (k - 1) * _CHUNK, _CHUNK)])
        lb = (_NCHUNK - 1) & 1
        cps[lb].wait()
        pltpu.sync_copy(
            bufs[lb],
            out_hbm.at[pl.ds(base + (_NCHUNK - 1) * _CHUNK, _CHUNK)])

    return _sc_gather


_K3_RB = 1024
_K3_STEPS = N_TOK // _K3_RB
_INV_N = 1.0 / float(N_TOK * D)


def _st_body(q_ref, xl_ref, xh_ref, olr_ref, ohr_ref, slr_ref, shr_ref):
    i = pl.program_id(0)
    xl = xl_ref[...]
    xh = xh_ref[...]
    dl = q_ref[:, :D] - xl
    dh = q_ref[:, D:] - xh
    olr_ref[...] = xl + dl
    ohr_ref[...] = xh + dh
    sl = jnp.sum(dl * dl).reshape(1, 1)
    sh = jnp.sum(dh * dh).reshape(1, 1)
    acc_l = jnp.where(i == 0, sl, slr_ref[...] + sl)
    acc_h = jnp.where(i == 0, sh, shr_ref[...] + sh)
    last = i == _K3_STEPS - 1
    slr_ref[...] = jnp.where(last, acc_l * _INV_N, acc_l)
    shr_ref[...] = jnp.where(last, acc_h * _INV_N, acc_h)


def _straight_through(q, xl, xh):
    row_spec = pl.BlockSpec((_K3_RB, DIM2), lambda i: (i, 0))
    half_spec = pl.BlockSpec((_K3_RB, D), lambda i: (i, 0))
    one_spec = pl.BlockSpec((1, 1), lambda i: (0, 0))
    return pl.pallas_call(
        _st_body,
        grid=(_K3_STEPS,),
        in_specs=[row_spec, half_spec, half_spec],
        out_specs=[half_spec, half_spec, one_spec, one_spec],
        out_shape=[
            jax.ShapeDtypeStruct((N_TOK, D), jnp.float32),
            jax.ShapeDtypeStruct((N_TOK, D), jnp.float32),
            jax.ShapeDtypeStruct((1, 1), jnp.float32),
            jax.ShapeDtypeStruct((1, 1), jnp.float32),
        ],
    )(q, xl, xh)


def kernel(input_hr, input_lr, embed_lr, embed_hr):
    dim = input_hr.shape[-1]
    flatten_hr = input_hr.reshape(-1, dim)
    flatten_lr = input_lr.reshape(-1, dim)
    flatten = jnp.concatenate([flatten_lr, flatten_hr], axis=1)
    embed = jnp.concatenate([embed_lr, embed_hr], axis=0)
    fsq = (flatten ** 2).sum(1, keepdims=True)
    esq = (embed ** 2).sum(0, keepdims=True)

    ind2d = _fused_argmin(flatten, fsq, embed, esq)

    idx_hbm = ind2d.reshape(N_TOK // _CHUNK, _CHUNK)
    q = _build_sc_gather()(idx_hbm, embed.T)

    o_lr, o_hr, s_lr, s_hr = _straight_through(q, flatten_lr, flatten_hr)

    embed_ind = ind2d.reshape(input_hr.shape[:-1])
    quantize_hr = o_hr.reshape(input_hr.shape)
    quantize_lr = o_lr.reshape(input_lr.shape)
    diff_hr = s_hr.reshape(())
    diff_lr = s_lr.reshape(())
    return (quantize_hr, quantize_lr, diff_hr, diff_lr, embed_ind, embed_ind)


# speedup vs baseline: 1.3424x; 1.0639x over previous
import functools

import jax
import jax.numpy as jnp
from jax import lax
from jax.experimental import pallas as pl
from jax.experimental.pallas import tpu as pltpu
from jax.experimental.pallas import tpu_sc as plsc

DIM2 = 512
N_CODES = 8192
N_TOK = 8192
BN = 512
LW = 256
RB = 1024
N_TILES = N_CODES // BN
D = 256
_HALF_TILES = N_TILES // 2


def _argmin_body(f_ref, fsq_ref, e_ref, esq_ref, out_ref,
                 rv_ref, ri_ref, c0v_ref, c0i_ref):
    n = pl.program_id(0)
    lane = lax.broadcasted_iota(jnp.int32, (1, LW), 1)

    e = e_ref[...].astype(jnp.bfloat16)
    esq = esq_ref[...]
    for mi in range(N_TOK // RB):
        rows = pl.ds(mi * RB, RB)
        f = (2.0 * f_ref[rows, :]).astype(jnp.bfloat16)
        mm = jnp.dot(f, e, preferred_element_type=jnp.float32)
        d = (fsq_ref[rows, :] - mm) + esq
        d0 = d[:, :LW]
        d1 = d[:, LW:]
        t1 = d1 < d0
        db = jnp.where(t1, d1, d0)
        ib = n * BN + jnp.where(t1, LW + lane, lane)
        fresh = (n == 0) | (n == _HALF_TILES)
        cur_v = jnp.where(fresh, jnp.inf, rv_ref[rows, :])
        cur_i = jnp.where(fresh, 0, ri_ref[rows, :])
        take = db < cur_v
        rv_ref[rows, :] = jnp.where(take, db, cur_v)
        ri_ref[rows, :] = jnp.where(take, ib, cur_i)

    @pl.when((n == _HALF_TILES - 1) | (n == N_TILES - 1))
    def _reduce_chunk():
        for mi in range(N_TOK // RB):
            rows = pl.ds(mi * RB, RB)
            rv = rv_ref[rows, :]
            ri = ri_ref[rows, :]
            m = jnp.min(rv, axis=1, keepdims=True)
            ie = jnp.min(jnp.where(rv == m, ri, N_CODES), axis=1,
                         keepdims=True)
            @pl.when(n == _HALF_TILES - 1)
            def _save0():
                c0v_ref[rows, :] = m
                c0i_ref[rows, :] = ie

            @pl.when(n == N_TILES - 1)
            def _emit():
                m0b = c0v_ref[rows, :].astype(jnp.bfloat16).astype(jnp.float32)
                out_ref[rows, :] = jnp.where(m < m0b, ie, c0i_ref[rows, :])


def _fused_argmin(flatten, fsq, embed, esq):
    return pl.pallas_call(
        _argmin_body,
        grid=(N_TILES,),
        in_specs=[
            pl.BlockSpec((N_TOK, DIM2), lambda n: (0, 0)),
            pl.BlockSpec((N_TOK, 1), lambda n: (0, 0)),
            pl.BlockSpec((DIM2, BN), lambda n: (0, n)),
            pl.BlockSpec((1, BN), lambda n: (0, n)),
        ],
        out_specs=pl.BlockSpec((N_TOK, 1), lambda n: (0, 0)),
        out_shape=jax.ShapeDtypeStruct((N_TOK, 1), jnp.int32),
        scratch_shapes=[
            pltpu.VMEM((N_TOK, LW), jnp.float32),
            pltpu.VMEM((N_TOK, LW), jnp.int32),
            pltpu.VMEM((N_TOK, 1), jnp.float32),
            pltpu.VMEM((N_TOK, 1), jnp.int32),
        ],
    )(flatten, fsq, embed, esq)


_CHUNK = 64
_NCHUNK = 4


@functools.lru_cache(maxsize=1)
def _build_sc_gather():
    mesh = plsc.VectorSubcoreMesh(core_axis_name="c", subcore_axis_name="s")

    @functools.partial(
        pl.kernel,
        out_type=jax.ShapeDtypeStruct((N_TOK, DIM2), jnp.float32),
        mesh=mesh,
        scratch_types=[
            pltpu.VMEM((_NCHUNK, _CHUNK), jnp.int32),
            pltpu.VMEM((_CHUNK, DIM2), jnp.float32),
            pltpu.VMEM((_CHUNK, DIM2), jnp.float32),
            pltpu.SemaphoreType.DMA,
            pltpu.SemaphoreType.DMA,
        ],
    )
    def _sc_gather(idx_hbm, tab_hbm, out_hbm,
                   idx_v, rows_a, rows_b, sem_a, sem_b):
        wid = lax.axis_index("s") * 2 + lax.axis_index("c")
        base = wid * (_NCHUNK * _CHUNK)
        pltpu.sync_copy(idx_hbm.at[pl.ds(wid * _NCHUNK, _NCHUNK)], idx_v)
        bufs = (rows_a, rows_b)
        sems = (sem_a, sem_b)
        cps = [None, None]
        for k in range(_NCHUNK):
            b = k & 1
            cps[b] = pltpu.async_copy(tab_hbm.at[idx_v.at[k]], bufs[b], sems[b])
            if k >= 1:
                pb = (k - 1) & 1
                cps[pb].wait()
                pltpu.sync_copy(
                    bufs[pb],
                    out_hbm.at[pl.ds(base + (k - 1) * _CHUNK, _CHUNK)])
        lb = (_NCHUNK - 1) & 1
        cps[lb].wait()
        pltpu.sync_copy(
            bufs[lb],
            out_hbm.at[pl.ds(base + (_NCHUNK - 1) * _CHUNK, _CHUNK)])

    return _sc_gather


_K3_RB = 1024
_K3_STEPS = N_TOK // _K3_RB
_INV_N = 1.0 / float(N_TOK * D)


def _st_body(q_ref, xl_ref, xh_ref, olr_ref, ohr_ref, slr_ref, shr_ref):
    i = pl.program_id(0)
    xl = xl_ref[...]
    xh = xh_ref[...]
    dl = q_ref[:, :D] - xl
    dh = q_ref[:, D:] - xh
    olr_ref[...] = xl + dl
    ohr_ref[...] = xh + dh
    sl = jnp.sum(dl * dl).reshape(1, 1)
    sh = jnp.sum(dh * dh).reshape(1, 1)
    acc_l = jnp.where(i == 0, sl, slr_ref[...] + sl)
    acc_h = jnp.where(i == 0, sh, shr_ref[...] + sh)
    last = i == _K3_STEPS - 1
    slr_ref[...] = jnp.where(last, acc_l * _INV_N, acc_l)
    shr_ref[...] = jnp.where(last, acc_h * _INV_N, acc_h)


def _straight_through(q, xl, xh):
    row_spec = pl.BlockSpec((_K3_RB, DIM2), lambda i: (i, 0))
    half_spec = pl.BlockSpec((_K3_RB, D), lambda i: (i, 0))
    one_spec = pl.BlockSpec((1, 1), lambda i: (0, 0))
    return pl.pallas_call(
        _st_body,
        grid=(_K3_STEPS,),
        in_specs=[row_spec, half_spec, half_spec],
        out_specs=[half_spec, half_spec, one_spec, one_spec],
        out_shape=[
            jax.ShapeDtypeStruct((N_TOK, D), jnp.float32),
            jax.ShapeDtypeStruct((N_TOK, D), jnp.float32),
            jax.ShapeDtypeStruct((1, 1), jnp.float32),
            jax.ShapeDtypeStruct((1, 1), jnp.float32),
        ],
    )(q, xl, xh)


def kernel(input_hr, input_lr, embed_lr, embed_hr):
    dim = input_hr.shape[-1]
    flatten_hr = input_hr.reshape(-1, dim)
    flatten_lr = input_lr.reshape(-1, dim)
    flatten = jnp.concatenate([flatten_lr, flatten_hr], axis=1)
    embed = jnp.concatenate([embed_lr, embed_hr], axis=0)
    fsq = (flatten ** 2).sum(1, keepdims=True)
    esq = (embed ** 2).sum(0, keepdims=True)

    ind2d = _fused_argmin(flatten, fsq, embed, esq)

    idx_hbm = ind2d.reshape(N_TOK // _CHUNK, _CHUNK)
    q = _build_sc_gather()(idx_hbm, embed.T)

    o_lr, o_hr, s_lr, s_hr = _straight_through(q, flatten_lr, flatten_hr)

    embed_ind = ind2d.reshape(input_hr.shape[:-1])
    quantize_hr = o_hr.reshape(input_hr.shape)
    quantize_lr = o_lr.reshape(input_lr.shape)
    diff_hr = s_hr.reshape(())
    diff_lr = s_lr.reshape(())
    return (quantize_hr, quantize_lr, diff_hr, diff_lr, embed_ind, embed_ind)


# speedup vs baseline: 1.3567x; 1.0106x over previous
import functools

import jax
import jax.numpy as jnp
from jax import lax
from jax.experimental import pallas as pl
from jax.experimental.pallas import tpu as pltpu
from jax.experimental.pallas import tpu_sc as plsc

DIM2 = 512
N_CODES = 8192
N_TOK = 8192
BN = 1024
LW = 256
RB = 1024
N_TILES = N_CODES // BN
D = 256
_HALF_TILES = N_TILES // 2


def _argmin_body(f_ref, fsq_ref, e_ref, esq_ref, out_ref,
                 rv_ref, ri_ref, c0v_ref, c0i_ref):
    n = pl.program_id(0)
    lane = lax.broadcasted_iota(jnp.int32, (1, LW), 1)

    e = e_ref[...].astype(jnp.bfloat16)
    esq = esq_ref[...]
    for mi in range(N_TOK // RB):
        rows = pl.ds(mi * RB, RB)
        f = (2.0 * f_ref[rows, :]).astype(jnp.bfloat16)
        mm = jnp.dot(f, e, preferred_element_type=jnp.float32)
        d = (fsq_ref[rows, :] - mm) + esq
        db = d[:, :LW]
        ib = n * BN + lane
        for k in range(1, BN // LW):
            dk = d[:, k * LW:(k + 1) * LW]
            tk = dk < db
            db = jnp.where(tk, dk, db)
            ib = jnp.where(tk, n * BN + k * LW + lane, ib)
        fresh = (n == 0) | (n == _HALF_TILES)
        cur_v = jnp.where(fresh, jnp.inf, rv_ref[rows, :])
        cur_i = jnp.where(fresh, 0, ri_ref[rows, :])
        take = db < cur_v
        rv_ref[rows, :] = jnp.where(take, db, cur_v)
        ri_ref[rows, :] = jnp.where(take, ib, cur_i)

    @pl.when((n == _HALF_TILES - 1) | (n == N_TILES - 1))
    def _reduce_chunk():
        for mi in range(N_TOK // RB):
            rows = pl.ds(mi * RB, RB)
            rv = rv_ref[rows, :]
            ri = ri_ref[rows, :]
            m = jnp.min(rv, axis=1, keepdims=True)
            ie = jnp.min(jnp.where(rv == m, ri, N_CODES), axis=1,
                         keepdims=True)
            @pl.when(n == _HALF_TILES - 1)
            def _save0():
                c0v_ref[rows, :] = m
                c0i_ref[rows, :] = ie

            @pl.when(n == N_TILES - 1)
            def _emit():
                m0b = c0v_ref[rows, :].astype(jnp.bfloat16).astype(jnp.float32)
                out_ref[rows, :] = jnp.where(m < m0b, ie, c0i_ref[rows, :])


def _fused_argmin(flatten, fsq, embed, esq):
    return pl.pallas_call(
        _argmin_body,
        grid=(N_TILES,),
        in_specs=[
            pl.BlockSpec((N_TOK, DIM2), lambda n: (0, 0)),
            pl.BlockSpec((N_TOK, 1), lambda n: (0, 0)),
            pl.BlockSpec((DIM2, BN), lambda n: (0, n)),
            pl.BlockSpec((1, BN), lambda n: (0, n)),
        ],
        out_specs=pl.BlockSpec((N_TOK, 1), lambda n: (0, 0)),
        out_shape=jax.ShapeDtypeStruct((N_TOK, 1), jnp.int32),
        scratch_shapes=[
            pltpu.VMEM((N_TOK, LW), jnp.float32),
            pltpu.VMEM((N_TOK, LW), jnp.int32),
            pltpu.VMEM((N_TOK, 1), jnp.float32),
            pltpu.VMEM((N_TOK, 1), jnp.int32),
        ],
    )(flatten, fsq, embed, esq)


_CHUNK = 64
_NCHUNK = 4


@functools.lru_cache(maxsize=1)
def _build_sc_gather():
    mesh = plsc.VectorSubcoreMesh(core_axis_name="c", subcore_axis_name="s")

    @functools.partial(
        pl.kernel,
        out_type=jax.ShapeDtypeStruct((N_TOK, DIM2), jnp.float32),
        mesh=mesh,
        scratch_types=[
            pltpu.VMEM((_NCHUNK, _CHUNK), jnp.int32),
            pltpu.VMEM((_CHUNK, DIM2), jnp.float32),
            pltpu.VMEM((_CHUNK, DIM2), jnp.float32),
            pltpu.SemaphoreType.DMA,
            pltpu.SemaphoreType.DMA,
        ],
    )
    def _sc_gather(idx_hbm, tab_hbm, out_hbm,
                   idx_v, rows_a, rows_b, sem_a, sem_b):
        wid = lax.axis_index("s") * 2 + lax.axis_index("c")
        base = wid * (_NCHUNK * _CHUNK)
        pltpu.sync_copy(idx_hbm.at[pl.ds(wid * _NCHUNK, _NCHUNK)], idx_v)
        bufs = (rows_a, rows_b)
        sems = (sem_a, sem_b)
        cps = [None, None]
        for k in range(_NCHUNK):
            b = k & 1
            cps[b] = pltpu.async_copy(tab_hbm.at[idx_v.at[k]], bufs[b], sems[b])
            if k >= 1:
                pb = (k - 1) & 1
                cps[pb].wait()
                pltpu.sync_copy(
                    bufs[pb],
                    out_hbm.at[pl.ds(base + (k - 1) * _CHUNK, _CHUNK)])
        lb = (_NCHUNK - 1) & 1
        cps[lb].wait()
        pltpu.sync_copy(
            bufs[lb],
            out_hbm.at[pl.ds(base + (_NCHUNK - 1) * _CHUNK, _CHUNK)])

    return _sc_gather


_K3_RB = 1024
_K3_STEPS = N_TOK // _K3_RB
_INV_N = 1.0 / float(N_TOK * D)


def _st_body(q_ref, xl_ref, xh_ref, olr_ref, ohr_ref, slr_ref, shr_ref):
    i = pl.program_id(0)
    xl = xl_ref[...]
    xh = xh_ref[...]
    dl = q_ref[:, :D] - xl
    dh = q_ref[:, D:] - xh
    olr_ref[...] = xl + dl
    ohr_ref[...] = xh + dh
    sl = jnp.sum(dl * dl).reshape(1, 1)
    sh = jnp.sum(dh * dh).reshape(1, 1)
    acc_l = jnp.where(i == 0, sl, slr_ref[...] + sl)
    acc_h = jnp.where(i == 0, sh, shr_ref[...] + sh)
    last = i == _K3_STEPS - 1
    slr_ref[...] = jnp.where(last, acc_l * _INV_N, acc_l)
    shr_ref[...] = jnp.where(last, acc_h * _INV_N, acc_h)


def _straight_through(q, xl, xh):
    row_spec = pl.BlockSpec((_K3_RB, DIM2), lambda i: (i, 0))
    half_spec = pl.BlockSpec((_K3_RB, D), lambda i: (i, 0))
    one_spec = pl.BlockSpec((1, 1), lambda i: (0, 0))
    return pl.pallas_call(
        _st_body,
        grid=(_K3_STEPS,),
        in_specs=[row_spec, half_spec, half_spec],
        out_specs=[half_spec, half_spec, one_spec, one_spec],
        out_shape=[
            jax.ShapeDtypeStruct((N_TOK, D), jnp.float32),
            jax.ShapeDtypeStruct((N_TOK, D), jnp.float32),
            jax.ShapeDtypeStruct((1, 1), jnp.float32),
            jax.ShapeDtypeStruct((1, 1), jnp.float32),
        ],
    )(q, xl, xh)


def kernel(input_hr, input_lr, embed_lr, embed_hr):
    dim = input_hr.shape[-1]
    flatten_hr = input_hr.reshape(-1, dim)
    flatten_lr = input_lr.reshape(-1, dim)
    flatten = jnp.concatenate([flatten_lr, flatten_hr], axis=1)
    embed = jnp.concatenate([embed_lr, embed_hr], axis=0)
    fsq = (flatten ** 2).sum(1, keepdims=True)
    esq = (embed ** 2).sum(0, keepdims=True)

    ind2d = _fused_argmin(flatten, fsq, embed, esq)

    idx_hbm = ind2d.reshape(N_TOK // _CHUNK, _CHUNK)
    q = _build_sc_gather()(idx_hbm, embed.T)

    o_lr, o_hr, s_lr, s_hr = _straight_through(q, flatten_lr, flatten_hr)

    embed_ind = ind2d.reshape(input_hr.shape[:-1])
    quantize_hr = o_hr.reshape(input_hr.shape)
    quantize_lr = o_lr.reshape(input_lr.shape)
    diff_hr = s_hr.reshape(())
    diff_lr = s_lr.reshape(())
    return (quantize_hr, quantize_lr, diff_hr, diff_lr, embed_ind, embed_ind)


# speedup vs baseline: 1.3606x; 1.0029x over previous
import functools

import jax
import jax.numpy as jnp
from jax import lax
from jax.experimental import pallas as pl
from jax.experimental.pallas import tpu as pltpu
from jax.experimental.pallas import tpu_sc as plsc

DIM2 = 512
N_CODES = 8192
N_TOK = 8192
BN = 1024
LW = 256
RB = 1024
N_TILES = N_CODES // BN
D = 256
_HALF_TILES = N_TILES // 2


def _argmin_body(f_ref, fsq_ref, e_ref, esq_ref, out_ref,
                 rv_ref, ri_ref, c0v_ref, c0i_ref):
    n = pl.program_id(0)
    lane = lax.broadcasted_iota(jnp.int32, (1, LW), 1)

    @pl.when((n == 0) | (n == _HALF_TILES))
    def _init():
        rv_ref[...] = jnp.full((N_TOK, LW), jnp.inf, jnp.float32)

    e = e_ref[...].astype(jnp.bfloat16)
    for mi in range(N_TOK // RB):
        rows = pl.ds(mi * RB, RB)
        f = (2.0 * f_ref[rows, :]).astype(jnp.bfloat16)
        mm = jnp.dot(f, e, preferred_element_type=jnp.float32)
        fsq = fsq_ref[rows, :]
        db = (fsq - mm[:, :LW]) + esq_ref[:, :LW]
        ib = n * BN + lane
        for k in range(1, BN // LW):
            dk = (fsq - mm[:, k * LW:(k + 1) * LW]) \
                + esq_ref[:, k * LW:(k + 1) * LW]
            tk = dk < db
            db = jnp.where(tk, dk, db)
            ib = jnp.where(tk, n * BN + k * LW + lane, ib)
        cur_v = rv_ref[rows, :]
        take = db < cur_v
        rv_ref[rows, :] = jnp.where(take, db, cur_v)
        ri_ref[rows, :] = jnp.where(take, ib, ri_ref[rows, :])

    @pl.when((n == _HALF_TILES - 1) | (n == N_TILES - 1))
    def _reduce_chunk():
        for mi in range(N_TOK // RB):
            rows = pl.ds(mi * RB, RB)
            rv = rv_ref[rows, :]
            ri = ri_ref[rows, :]
            m = jnp.min(rv, axis=1, keepdims=True)
            ie = jnp.min(jnp.where(rv == m, ri, N_CODES), axis=1,
                         keepdims=True)
            @pl.when(n == _HALF_TILES - 1)
            def _save0():
                c0v_ref[rows, :] = m
                c0i_ref[rows, :] = ie

            @pl.when(n == N_TILES - 1)
            def _emit():
                m0b = c0v_ref[rows, :].astype(jnp.bfloat16).astype(jnp.float32)
                out_ref[rows, :] = jnp.where(m < m0b, ie, c0i_ref[rows, :])


def _fused_argmin(flatten, fsq, embed, esq):
    return pl.pallas_call(
        _argmin_body,
        grid=(N_TILES,),
        in_specs=[
            pl.BlockSpec((N_TOK, DIM2), lambda n: (0, 0)),
            pl.BlockSpec((N_TOK, 1), lambda n: (0, 0)),
            pl.BlockSpec((DIM2, BN), lambda n: (0, n)),
            pl.BlockSpec((1, BN), lambda n: (0, n)),
        ],
        out_specs=pl.BlockSpec((N_TOK, 1), lambda n: (0, 0)),
        out_shape=jax.ShapeDtypeStruct((N_TOK, 1), jnp.int32),
        scratch_shapes=[
            pltpu.VMEM((N_TOK, LW), jnp.float32),
            pltpu.VMEM((N_TOK, LW), jnp.int32),
            pltpu.VMEM((N_TOK, 1), jnp.float32),
            pltpu.VMEM((N_TOK, 1), jnp.int32),
        ],
    )(flatten, fsq, embed, esq)


_CHUNK = 64
_NCHUNK = 4


@functools.lru_cache(maxsize=1)
def _build_sc_gather():
    mesh = plsc.VectorSubcoreMesh(core_axis_name="c", subcore_axis_name="s")

    @functools.partial(
        pl.kernel,
        out_type=jax.ShapeDtypeStruct((N_TOK, DIM2), jnp.float32),
        mesh=mesh,
        scratch_types=[
            pltpu.VMEM((_NCHUNK, _CHUNK), jnp.int32),
            pltpu.VMEM((_CHUNK, DIM2), jnp.float32),
            pltpu.VMEM((_CHUNK, DIM2), jnp.float32),
            pltpu.SemaphoreType.DMA,
            pltpu.SemaphoreType.DMA,
        ],
    )
    def _sc_gather(idx_hbm, tab_hbm, out_hbm,
                   idx_v, rows_a, rows_b, sem_a, sem_b):
        wid = lax.axis_index("s") * 2 + lax.axis_index("c")
        base = wid * (_NCHUNK * _CHUNK)
        pltpu.sync_copy(idx_hbm.at[pl.ds(wid * _NCHUNK, _NCHUNK)], idx_v)
        bufs = (rows_a, rows_b)
        sems = (sem_a, sem_b)
        cps = [None, None]
        for k in range(_NCHUNK):
            b = k & 1
            cps[b] = pltpu.async_copy(tab_hbm.at[idx_v.at[k]], bufs[b], sems[b])
            if k >= 1:
                pb = (k - 1) & 1
                cps[pb].wait()
                pltpu.sync_copy(
                    bufs[pb],
                    out_hbm.at[pl.ds(base + (k - 1) * _CHUNK, _CHUNK)])
        lb = (_NCHUNK - 1) & 1
        cps[lb].wait()
        pltpu.sync_copy(
            bufs[lb],
            out_hbm.at[pl.ds(base + (_NCHUNK - 1) * _CHUNK, _CHUNK)])

    return _sc_gather


_K3_RB = 1024
_K3_STEPS = N_TOK // _K3_RB
_INV_N = 1.0 / float(N_TOK * D)


def _st_body(q_ref, xl_ref, xh_ref, olr_ref, ohr_ref, slr_ref, shr_ref):
    i = pl.program_id(0)
    xl = xl_ref[...]
    xh = xh_ref[...]
    dl = q_ref[:, :D] - xl
    dh = q_ref[:, D:] - xh
    olr_ref[...] = xl + dl
    ohr_ref[...] = xh + dh
    sl = jnp.sum(dl * dl).reshape(1, 1)
    sh = jnp.sum(dh * dh).reshape(1, 1)
    acc_l = jnp.where(i == 0, sl, slr_ref[...] + sl)
    acc_h = jnp.where(i == 0, sh, shr_ref[...] + sh)
    last = i == _K3_STEPS - 1
    slr_ref[...] = jnp.where(last, acc_l * _INV_N, acc_l)
    shr_ref[...] = jnp.where(last, acc_h * _INV_N, acc_h)


def _straight_through(q, xl, xh):
    row_spec = pl.BlockSpec((_K3_RB, DIM2), lambda i: (i, 0))
    half_spec = pl.BlockSpec((_K3_RB, D), lambda i: (i, 0))
    one_spec = pl.BlockSpec((1, 1), lambda i: (0, 0))
    return pl.pallas_call(
        _st_body,
        grid=(_K3_STEPS,),
        in_specs=[row_spec, half_spec, half_spec],
        out_specs=[half_spec, half_spec, one_spec, one_spec],
        out_shape=[
            jax.ShapeDtypeStruct((N_TOK, D), jnp.float32),
            jax.ShapeDtypeStruct((N_TOK, D), jnp.float32),
            jax.ShapeDtypeStruct((1, 1), jnp.float32),
            jax.ShapeDtypeStruct((1, 1), jnp.float32),
        ],
    )(q, xl, xh)


def kernel(input_hr, input_lr, embed_lr, embed_hr):
    dim = input_hr.shape[-1]
    flatten_hr = input_hr.reshape(-1, dim)
    flatten_lr = input_lr.reshape(-1, dim)
    flatten = jnp.concatenate([flatten_lr, flatten_hr], axis=1)
    embed = jnp.concatenate([embed_lr, embed_hr], axis=0)
    fsq = (flatten ** 2).sum(1, keepdims=True)
    esq = (embed ** 2).sum(0, keepdims=True)

    ind2d = _fused_argmin(flatten, fsq, embed, esq)

    idx_hbm = ind2d.reshape(N_TOK // _CHUNK, _CHUNK)
    q = _build_sc_gather()(idx_hbm, embed.T)

    o_lr, o_hr, s_lr, s_hr = _straight_through(q, flatten_lr, flatten_hr)

    embed_ind = ind2d.reshape(input_hr.shape[:-1])
    quantize_hr = o_hr.reshape(input_hr.shape)
    quantize_lr = o_lr.reshape(input_lr.shape)
    diff_hr = s_hr.reshape(())
    diff_lr = s_lr.reshape(())
    return (quantize_hr, quantize_lr, diff_hr, diff_lr, embed_ind, embed_ind)
